# baseline (speedup 1.0000x reference)
import jax
import jax.numpy as jnp
from jax.experimental import pallas as pl
from jax.experimental.pallas import tpu as pltpu

EMBED_DIM = 64
FFN_DIM = 128
NUM_EXPERTS = 4


def _moe_kernel(x_ref, wg_ref, w1_ref, b1_ref, w2_ref, b2_ref, ex_ref, o_ref):
    xb = x_ref[:]
    lT = jax.lax.dot_general(
        wg_ref[:], xb, (((1,), (1,)), ((), ())),
        preferred_element_type=jnp.float32)

    e_iota = jax.lax.broadcasted_iota(jnp.int32, lT.shape, 0)
    m1 = jnp.max(lT, axis=0, keepdims=True)
    idx1 = jnp.min(jnp.where(lT == m1, e_iota, NUM_EXPERTS),
                   axis=0, keepdims=True)
    masked = jnp.where(e_iota == idx1, -jnp.inf, lT)
    m2 = jnp.max(masked, axis=0, keepdims=True)
    idx2 = jnp.min(jnp.where(masked == m2, e_iota, NUM_EXPERTS),
                   axis=0, keepdims=True)
    p1 = 1.0 / (1.0 + jnp.exp(m2 - m1))
    p2 = 1.0 - p1
    wT = (jnp.where(e_iota == idx1, p1, 0.0)
          + jnp.where(e_iota == idx2, p2, 0.0))

    h = jax.lax.dot_general(
        xb, w1_ref[:], (((1,), (0,)), ((), ())),
        preferred_element_type=jnp.float32) + b1_ref[:]
    h = jnp.maximum(h, 0.0)

    out_all = jax.lax.dot_general(
        h, w2_ref[:], (((1,), (0,)), ((), ())),
        preferred_element_type=jnp.float32)

    wcol = jax.lax.dot_general(
        wT, ex_ref[:], (((0,), (0,)), ((), ())),
        preferred_element_type=jnp.float32)
    scaled = (out_all + b2_ref[:]) * wcol

    out = (scaled[:, 0:EMBED_DIM] + scaled[:, EMBED_DIM:2 * EMBED_DIM]
           + scaled[:, 2 * EMBED_DIM:3 * EMBED_DIM]
           + scaled[:, 3 * EMBED_DIM:4 * EMBED_DIM])
    o_ref[:] = out


def kernel(x, Wg, W1, b1, W2, b2):
    x = x.reshape(-1, x.shape[-1])
    T, D = x.shape
    E, _, F = W1.shape
    w1_cat = W1.transpose(1, 0, 2).reshape(D, E * F)
    b1_cat = b1.reshape(1, E * F)
    w2_bd = jnp.zeros((E * F, E * D), jnp.float32)
    for e in range(E):
        w2_bd = w2_bd.at[e * F:(e + 1) * F, e * D:(e + 1) * D].set(W2[e])
    expand = jnp.repeat(jnp.eye(E, dtype=jnp.float32), D, axis=1)

    TB = 1024
    grid = (T // TB,)
    out = pl.pallas_call(
        _moe_kernel,
        grid=grid,
        in_specs=[
            pl.BlockSpec((TB, D), lambda i: (i, 0)),
            pl.BlockSpec((E, D), lambda i: (0, 0)),
            pl.BlockSpec((D, E * F), lambda i: (0, 0)),
            pl.BlockSpec((1, E * F), lambda i: (0, 0)),
            pl.BlockSpec((E * F, E * D), lambda i: (0, 0)),
            pl.BlockSpec((1, E * D), lambda i: (0, 0)),
            pl.BlockSpec((E, E * D), lambda i: (0, 0)),
        ],
        out_specs=pl.BlockSpec((TB, D), lambda i: (i, 0)),
        out_shape=jax.ShapeDtypeStruct((T, D), jnp.float32),
        compiler_params=pltpu.CompilerParams(
            dimension_semantics=("parallel",)),
    )(x, Wg.T, w1_cat, b1_cat, w2_bd, b2.reshape(1, E * D), expand)
    return out


# speedup vs baseline: 1.0747x; 1.0747x over previous
import jax
import jax.numpy as jnp
from jax.experimental import pallas as pl
from jax.experimental.pallas import tpu as pltpu

EMBED_DIM = 64
FFN_DIM = 128
NUM_EXPERTS = 4


def _moe_kernel(x_ref, wg_ref, w1_ref, b1_ref, w2_ref, b2_ref, o_ref,
                w1s, w2s, b1s, b2s):
    D, F, E = EMBED_DIM, FFN_DIM, NUM_EXPERTS

    @pl.when(pl.program_id(0) == 0)
    def _prep():
        w2s[:] = jnp.zeros((E * F, E * D), jnp.float32)
        for e in range(E):
            w1s[:, e * F:(e + 1) * F] = w1_ref[e]
            w2s[e * F:(e + 1) * F, e * D:(e + 1) * D] = w2_ref[e]
            b1s[0:1, e * F:(e + 1) * F] = b1_ref[e:e + 1, :]
            b2s[0:1, e * D:(e + 1) * D] = b2_ref[e:e + 1, :]

    xb = x_ref[:]
    lT = jax.lax.dot_general(
        wg_ref[:], xb, (((0,), (1,)), ((), ())),
        preferred_element_type=jnp.float32)

    e_iota = jax.lax.broadcasted_iota(jnp.int32, lT.shape, 0)
    m1 = jnp.max(lT, axis=0, keepdims=True)
    idx1 = jnp.min(jnp.where(lT == m1, e_iota, E), axis=0, keepdims=True)
    masked = jnp.where(e_iota == idx1, -jnp.inf, lT)
    m2 = jnp.max(masked, axis=0, keepdims=True)
    idx2 = jnp.min(jnp.where(masked == m2, e_iota, E), axis=0, keepdims=True)
    p1 = 1.0 / (1.0 + jnp.exp(m2 - m1))
    p2 = 1.0 - p1
    wT = (jnp.where(e_iota == idx1, p1, 0.0)
          + jnp.where(e_iota == idx2, p2, 0.0))

    h = jax.lax.dot_general(
        xb, w1s[:], (((1,), (0,)), ((), ())),
        preferred_element_type=jnp.float32) + b1s[:]
    h = jnp.maximum(h, 0.0)

    out_all = jax.lax.dot_general(
        h, w2s[:], (((1,), (0,)), ((), ())),
        preferred_element_type=jnp.float32)

    lane_e = jax.lax.broadcasted_iota(jnp.int32, (E, E * D), 1) // D
    sub_e = jax.lax.broadcasted_iota(jnp.int32, (E, E * D), 0)
    ex = jnp.where(lane_e == sub_e, 1.0, 0.0).astype(jnp.float32)
    wcol = jax.lax.dot_general(
        wT, ex, (((0,), (0,)), ((), ())),
        preferred_element_type=jnp.float32)
    scaled = (out_all + b2s[:]) * wcol

    o_ref[:] = (scaled[:, 0:D] + scaled[:, D:2 * D]
                + scaled[:, 2 * D:3 * D] + scaled[:, 3 * D:4 * D])


def kernel(x, Wg, W1, b1, W2, b2):
    x = x.reshape(-1, x.shape[-1])
    T, D = x.shape
    E, _, F = W1.shape

    TB = 1024
    grid = (T // TB,)
    out = pl.pallas_call(
        _moe_kernel,
        grid=grid,
        in_specs=[
            pl.BlockSpec((TB, D), lambda i: (i, 0)),
            pl.BlockSpec((D, E), lambda i: (0, 0)),
            pl.BlockSpec((E, D, F), lambda i: (0, 0, 0)),
            pl.BlockSpec((E, F), lambda i: (0, 0)),
            pl.BlockSpec((E, F, D), lambda i: (0, 0, 0)),
            pl.BlockSpec((E, D), lambda i: (0, 0)),
        ],
        out_specs=pl.BlockSpec((TB, D), lambda i: (i, 0)),
        out_shape=jax.ShapeDtypeStruct((T, D), jnp.float32),
        scratch_shapes=[
            pltpu.VMEM((D, E * F), jnp.float32),
            pltpu.VMEM((E * F, E * D), jnp.float32),
            pltpu.VMEM((1, E * F), jnp.float32),
            pltpu.VMEM((1, E * D), jnp.float32),
        ],
        compiler_params=pltpu.CompilerParams(
            dimension_semantics=("arbitrary",)),
    )(x, Wg, W1, b1, W2, b2)
    return out
